# Initial kernel scaffold; baseline (speedup 1.0000x reference)
import jax
import jax.numpy as jnp
from jax.experimental import pallas as pl


def kernel(hidden_states, gate_w, e_bias, W1, W2, W3):
    raise NotImplementedError("write your pallas kernel here")


# speedup vs baseline: 4.6627x; 4.6627x over previous
import jax
import jax.numpy as jnp
from jax.experimental import pallas as pl
from jax.experimental.pallas import tpu as pltpu

E = 64
TOP_K = 2
H = 1024
I = 1024
S = 2048
T = 64


def _router_body(x_ref, gw_ref, eb_ref, wf_ref, rk_ref, ut_ref):
    logits = jax.lax.dot_general(
        gw_ref[...], x_ref[...],
        dimension_numbers=(((1,), (1,)), ((), ())),
        preferred_element_type=jnp.float32)
    rw = jax.nn.sigmoid(logits)
    scores = rw + eb_ref[:, 0:1]
    esub = jax.lax.broadcasted_iota(jnp.int32, (E, S), 0)
    m1 = jnp.max(scores, axis=0, keepdims=True)
    a1 = jnp.min(jnp.where(scores == m1, esub, E), axis=0, keepdims=True)
    sel1 = esub == a1
    masked = jnp.where(sel1, -jnp.inf, scores)
    m2 = jnp.max(masked, axis=0, keepdims=True)
    a2 = jnp.min(jnp.where(masked == m2, esub, E), axis=0, keepdims=True)
    sel2 = esub == a2
    w1 = jnp.sum(jnp.where(sel1, rw, 0.0), axis=0, keepdims=True)
    w2 = jnp.sum(jnp.where(sel2, rw, 0.0), axis=0, keepdims=True)
    s = w1 + w2
    wf_ref[...] = (jnp.where(sel1, w1 / s, 0.0)
                   + jnp.where(sel2, w2 / s, 0.0)).astype(jnp.float32)

    C = 512
    for j0 in range(0, S, C):
        ii = jax.lax.broadcasted_iota(jnp.int32, (S, C), 0)
        jj = jax.lax.broadcasted_iota(jnp.int32, (S, C), 1) + j0
        ut_ref[:, j0:j0 + C] = (ii <= jj).astype(jnp.float32)

    sel = (sel1 | sel2).astype(jnp.float32)
    rk_ref[...] = jax.lax.dot_general(
        sel, ut_ref[...], dimension_numbers=(((1,), (0,)), ((), ())),
        preferred_element_type=jnp.float32) - 1.0


def _ffn_body(x_ref, wf_ref, rk_ref, W1_ref, W2_ref, W3_ref, out_ref):
    e = pl.program_id(0)

    @pl.when(e == 0)
    def _():
        out_ref[...] = jnp.zeros(out_ref.shape, out_ref.dtype)

    w = wf_ref[0]
    m = w != 0.0
    rank = rk_ref[0]
    cnt = (rank[0, S - 1] + 1.0).astype(jnp.int32)
    ntiles = (cnt + T - 1) // T

    x = x_ref[...]
    W1 = W1_ref[0]
    W2 = W2_ref[0]
    W3 = W3_ref[0]

    def tile_body(t, carry):
        base = (t * T).astype(jnp.float32)
        rows = jax.lax.broadcasted_iota(jnp.int32, (T, S), 0).astype(jnp.float32)
        G = jnp.where((rank - base == rows) & m, 1.0, 0.0)
        xs = jax.lax.dot_general(
            G, x, dimension_numbers=(((1,), (0,)), ((), ())),
            preferred_element_type=jnp.float32)
        a = jax.lax.dot_general(
            xs, W1, dimension_numbers=(((1,), (1,)), ((), ())),
            preferred_element_type=jnp.float32)
        b = jax.lax.dot_general(
            xs, W3, dimension_numbers=(((1,), (1,)), ((), ())),
            preferred_element_type=jnp.float32)
        h = a * jax.nn.sigmoid(a) * b
        y = jax.lax.dot_general(
            h, W2, dimension_numbers=(((1,), (1,)), ((), ())),
            preferred_element_type=jnp.float32)
        wrow = jnp.sum(G * w, axis=1, keepdims=True)
        out_ref[...] += jax.lax.dot_general(
            G, y * wrow, dimension_numbers=(((0,), (0,)), ((), ())),
            preferred_element_type=jnp.float32)
        return carry

    jax.lax.fori_loop(0, ntiles, tile_body, 0)


def kernel(hidden_states, gate_w, e_bias, W1, W2, W3):
    b, s, h = hidden_states.shape
    x = hidden_states.reshape(s, h)
    eb = jnp.broadcast_to(e_bias[:, None], (E, 128))

    wfull, rank = pl.pallas_call(
        _router_body,
        out_shape=(
            jax.ShapeDtypeStruct((E, S), jnp.float32),
            jax.ShapeDtypeStruct((E, S), jnp.float32),
        ),
        in_specs=[
            pl.BlockSpec((S, H), lambda: (0, 0)),
            pl.BlockSpec((E, H), lambda: (0, 0)),
            pl.BlockSpec((E, 128), lambda: (0, 0)),
        ],
        out_specs=(
            pl.BlockSpec((E, S), lambda: (0, 0)),
            pl.BlockSpec((E, S), lambda: (0, 0)),
        ),
        scratch_shapes=[pltpu.VMEM((S, S), jnp.float32)],
    )(x, gate_w, eb)

    wf3 = wfull.reshape(E, 1, S)
    rk3 = rank.reshape(E, 1, S)

    out = pl.pallas_call(
        _ffn_body,
        grid=(E,),
        out_shape=jax.ShapeDtypeStruct((S, H), jnp.float32),
        in_specs=[
            pl.BlockSpec((S, H), lambda e: (0, 0)),
            pl.BlockSpec((1, 1, S), lambda e: (e, 0, 0)),
            pl.BlockSpec((1, 1, S), lambda e: (e, 0, 0)),
            pl.BlockSpec((1, I, H), lambda e: (e, 0, 0)),
            pl.BlockSpec((1, H, I), lambda e: (e, 0, 0)),
            pl.BlockSpec((1, I, H), lambda e: (e, 0, 0)),
        ],
        out_specs=pl.BlockSpec((S, H), lambda e: (0, 0)),
        compiler_params=pltpu.CompilerParams(
            dimension_semantics=("arbitrary",),
            vmem_limit_bytes=66060288),
    )(x, wf3, rk3, W1, W2, W3)

    return out.reshape(b, s, h)
